# baseline (speedup 1.0000x reference)
import jax
import jax.numpy as jnp
from jax.experimental import pallas as pl
from jax.experimental.pallas import tpu as pltpu


def _basic_block(x, w1, gamma1, beta1, w2, gamma2, beta2, eps=1e-5):
    N, C, H, W = x.shape
    Cout = w1.shape[0]
    f32 = jnp.float32
    bf16 = jnp.bfloat16
    HW = H * W
    P = W + 1
    L = HW + 2 * P
    K9 = 9 * C
    offs = [kh * W + kw for kh in range(3) for kw in range(3)]
    cnt = float(N * H * W)

    xf = x.reshape(N, C, HW)
    w1s = jnp.transpose(w1, (0, 2, 3, 1)).reshape(Cout, K9).astype(bf16)
    w2s = jnp.transpose(w2, (0, 2, 3, 1)).reshape(Cout, K9).astype(bf16)
    g1 = gamma1.reshape(Cout, 1).astype(f32)
    b1 = beta1.reshape(Cout, 1).astype(f32)
    g2 = gamma2.reshape(Cout, 1).astype(f32)
    b2 = beta2.reshape(Cout, 1).astype(f32)
    lane = jnp.arange(HW, dtype=jnp.int32) % W
    m0 = (lane != 0).astype(bf16).reshape(1, HW)
    m2 = (lane != W - 1).astype(bf16).reshape(1, HW)

    def _tree_sum(v):
        n = v.shape[0]
        while n > 1:
            h = n // 2
            v = v[:h] + v[h:2 * h] if 2 * h == n else \
                jnp.concatenate([v[:h] + v[h:2 * h], v[2 * h:]], axis=0)
            n = v.shape[0]
        return v[0]

    def _fold(ssum_ref, ssq_ref, g_ref, bt_ref):
        mean = _tree_sum(ssum_ref[...]) / cnt
        var = jnp.maximum(_tree_sum(ssq_ref[...]) / cnt - mean * mean, 0.0)
        scale = g_ref[...] / jnp.sqrt(var + eps)
        shift = bt_ref[...] - mean * scale
        return scale, shift

    def _stack_taps(slab_ref, xs_ref, m0v, m2v):
        for k, off in enumerate(offs):
            src = slab_ref[:, off:off + HW]
            if k % 3 == 0:
                src = src * m0v
            elif k % 3 == 2:
                src = src * m2v
            xs_ref[k * C:(k + 1) * C, :] = src

    B = 2
    G = N // B

    def p1_kernel(x_ref, w1s_ref, m0_ref, m2_ref, y1_ref, ssum_ref, ssq_ref,
                  slab_ref, xs_ref):
        for i in range(B):
            slab_ref[:, :P] = jnp.zeros((C, P), bf16)
            slab_ref[:, P + HW:] = jnp.zeros((C, L - P - HW), bf16)
            slab_ref[:, P:P + HW] = x_ref[i].astype(bf16)
            _stack_taps(slab_ref, xs_ref, m0_ref[...], m2_ref[...])
            y1 = jnp.dot(w1s_ref[...], xs_ref[...],
                         preferred_element_type=f32)
            ssum_ref[i] = jnp.sum(y1, axis=1, keepdims=True)
            ssq_ref[i] = jnp.sum(y1 * y1, axis=1, keepdims=True)
            y1_ref[i] = y1.astype(bf16)

    def p2_kernel(y1b_ref, w2s_ref, m0_ref, m2_ref, ssum1_ref, ssq1_ref,
                  g1_ref, b1_ref, y2_ref, ssum_ref, ssq_ref,
                  slab_ref, as_ref):
        s1, sh1 = _fold(ssum1_ref, ssq1_ref, g1_ref, b1_ref)
        for i in range(B):
            a1 = jnp.maximum(y1b_ref[i].astype(f32) * s1 + sh1, 0.0)
            slab_ref[:, :P] = jnp.zeros((Cout, P), bf16)
            slab_ref[:, P + HW:] = jnp.zeros((Cout, L - P - HW), bf16)
            slab_ref[:, P:P + HW] = a1.astype(bf16)
            _stack_taps(slab_ref, as_ref, m0_ref[...], m2_ref[...])
            y2 = jnp.dot(w2s_ref[...], as_ref[...],
                         preferred_element_type=f32)
            ssum_ref[i] = jnp.sum(y2, axis=1, keepdims=True)
            ssq_ref[i] = jnp.sum(y2 * y2, axis=1, keepdims=True)
            y2_ref[i] = y2.astype(bf16)

    def p3_kernel(y2b_ref, x_ref, ssum2_ref, ssq2_ref, g2_ref, b2_ref, o_ref):
        s2, sh2 = _fold(ssum2_ref, ssq2_ref, g2_ref, b2_ref)
        for i in range(B):
            o_ref[i] = jnp.maximum(
                y2b_ref[i].astype(f32) * s2 + sh2 + x_ref[i], 0.0)

    xf_spec = pl.BlockSpec((B, C, HW), lambda b: (b, 0, 0))
    ws_spec = pl.BlockSpec((Cout, K9), lambda b: (0, 0))
    mask_spec = pl.BlockSpec((1, HW), lambda b: (0, 0))
    vec_spec = pl.BlockSpec((Cout, 1), lambda b: (0, 0))
    part_spec = pl.BlockSpec((B, Cout, 1), lambda b: (b, 0, 0))
    partfull_spec = pl.BlockSpec((N, Cout, 1), lambda b: (0, 0, 0))
    act_spec = pl.BlockSpec((B, Cout, HW), lambda b: (b, 0, 0))
    part_shape = jax.ShapeDtypeStruct((N, Cout, 1), f32)
    act_shape = jax.ShapeDtypeStruct((N, Cout, HW), bf16)
    parallel = pltpu.CompilerParams(dimension_semantics=("arbitrary",))

    y1b, ssum1, ssq1 = pl.pallas_call(
        p1_kernel,
        out_shape=(act_shape, part_shape, part_shape),
        grid=(G,),
        in_specs=[xf_spec, ws_spec, mask_spec, mask_spec],
        out_specs=(act_spec, part_spec, part_spec),
        scratch_shapes=[pltpu.VMEM((C, L), bf16), pltpu.VMEM((K9, HW), bf16)],
        compiler_params=parallel,
    )(xf, w1s, m0, m2)

    y2b, ssum2, ssq2 = pl.pallas_call(
        p2_kernel,
        out_shape=(act_shape, part_shape, part_shape),
        grid=(G,),
        in_specs=[act_spec, ws_spec, mask_spec, mask_spec,
                  partfull_spec, partfull_spec, vec_spec, vec_spec],
        out_specs=(act_spec, part_spec, part_spec),
        scratch_shapes=[pltpu.VMEM((Cout, L), bf16),
                        pltpu.VMEM((K9, HW), bf16)],
        compiler_params=parallel,
    )(y1b, w2s, m0, m2, ssum1, ssq1, g1, b1)

    out = pl.pallas_call(
        p3_kernel,
        out_shape=jax.ShapeDtypeStruct((N, Cout, HW), x.dtype),
        grid=(G,),
        in_specs=[act_spec, xf_spec, partfull_spec, partfull_spec,
                  vec_spec, vec_spec],
        out_specs=pl.BlockSpec((B, Cout, HW), lambda b: (b, 0, 0)),
        compiler_params=parallel,
    )(y2b, xf, ssum2, ssq2, g2, b2)

    return out.reshape(N, Cout, H, W)


def kernel(x, w1, gamma1, beta1, w2, gamma2, beta2):
    return _basic_block(x, w1, gamma1, beta1, w2, gamma2, beta2)


# speedup vs baseline: 1.0321x; 1.0321x over previous
import jax
import jax.numpy as jnp
from jax.experimental import pallas as pl
from jax.experimental.pallas import tpu as pltpu


def _basic_block(x, w1, gamma1, beta1, w2, gamma2, beta2, eps=1e-5):
    N, C, H, W = x.shape
    Cout = w1.shape[0]
    f32 = jnp.float32
    bf16 = jnp.bfloat16
    HW = H * W
    P = W + 1
    L = HW + 2 * P
    K9 = 9 * C
    offs = [kh * W + kw for kh in range(3) for kw in range(3)]
    cnt = float(N * H * W)

    xf = x.reshape(N, C, HW)
    w1s = jnp.transpose(w1, (0, 2, 3, 1)).reshape(Cout, K9).astype(bf16)
    w2s = jnp.transpose(w2, (0, 2, 3, 1)).reshape(Cout, K9).astype(bf16)
    g1 = gamma1.reshape(Cout, 1).astype(f32)
    b1 = beta1.reshape(Cout, 1).astype(f32)
    g2 = gamma2.reshape(Cout, 1).astype(f32)
    b2 = beta2.reshape(Cout, 1).astype(f32)
    lane = jnp.arange(HW, dtype=jnp.int32) % W
    mL = (lane != W - 1).astype(f32).reshape(1, HW)
    mR = (lane != 0).astype(f32).reshape(1, HW)

    def _tree_sum(v):
        n = v.shape[0]
        while n > 1:
            h = n // 2
            v = v[:h] + v[h:2 * h] if 2 * h == n else \
                jnp.concatenate([v[:h] + v[h:2 * h], v[2 * h:]], axis=0)
            n = v.shape[0]
        return v[0]

    def _fold(ssum_ref, ssq_ref, g_ref, bt_ref):
        mean = _tree_sum(ssum_ref[...]) / cnt
        var = jnp.maximum(_tree_sum(ssq_ref[...]) / cnt - mean * mean, 0.0)
        scale = g_ref[...] / jnp.sqrt(var + eps)
        shift = bt_ref[...] - mean * scale
        return scale, shift

    B = 2
    G = N // B

    def _zero_borders(b, refs):
        @pl.when(b == 0)
        def _():
            for r in refs:
                r[:, :P] = jnp.zeros((C, P), bf16)
                r[:, P + HW:] = jnp.zeros((C, L - P - HW), bf16)

    def _fill_slabs(v, mLv, mRv, slab_ref, slabL_ref, slabR_ref):
        slab_ref[:, P:P + HW] = v.astype(bf16)
        slabL_ref[:, P:P + HW] = (v * mLv).astype(bf16)
        slabR_ref[:, P:P + HW] = (v * mRv).astype(bf16)

    def _stack_taps(slab_ref, slabL_ref, slabR_ref, xs_ref):
        for k, off in enumerate(offs):
            src = (slabL_ref, slab_ref, slabR_ref)[k % 3]
            xs_ref[k * C:(k + 1) * C, :] = src[:, off:off + HW]

    def p1_kernel(x_ref, w1s_ref, mL_ref, mR_ref, y1_ref, ssum_ref, ssq_ref,
                  slab_ref, slabL_ref, slabR_ref, xs_ref):
        _zero_borders(pl.program_id(0), (slab_ref, slabL_ref, slabR_ref))
        for i in range(B):
            _fill_slabs(x_ref[i], mL_ref[...], mR_ref[...],
                        slab_ref, slabL_ref, slabR_ref)
            _stack_taps(slab_ref, slabL_ref, slabR_ref, xs_ref)
            y1 = jnp.dot(w1s_ref[...], xs_ref[...],
                         preferred_element_type=f32)
            ssum_ref[i] = jnp.sum(y1, axis=1, keepdims=True)
            ssq_ref[i] = jnp.sum(y1 * y1, axis=1, keepdims=True)
            y1_ref[i] = y1.astype(bf16)

    def p2_kernel(y1b_ref, w2s_ref, mL_ref, mR_ref, ssum1_ref, ssq1_ref,
                  g1_ref, b1_ref, y2_ref, ssum_ref, ssq_ref,
                  slab_ref, slabL_ref, slabR_ref, as_ref):
        _zero_borders(pl.program_id(0), (slab_ref, slabL_ref, slabR_ref))
        s1, sh1 = _fold(ssum1_ref, ssq1_ref, g1_ref, b1_ref)
        for i in range(B):
            a1 = jnp.maximum(y1b_ref[i].astype(f32) * s1 + sh1, 0.0)
            _fill_slabs(a1, mL_ref[...], mR_ref[...],
                        slab_ref, slabL_ref, slabR_ref)
            _stack_taps(slab_ref, slabL_ref, slabR_ref, as_ref)
            y2 = jnp.dot(w2s_ref[...], as_ref[...],
                         preferred_element_type=f32)
            ssum_ref[i] = jnp.sum(y2, axis=1, keepdims=True)
            ssq_ref[i] = jnp.sum(y2 * y2, axis=1, keepdims=True)
            y2_ref[i] = y2.astype(bf16)

    def p3_kernel(y2b_ref, x_ref, ssum2_ref, ssq2_ref, g2_ref, b2_ref, o_ref):
        s2, sh2 = _fold(ssum2_ref, ssq2_ref, g2_ref, b2_ref)
        for i in range(B):
            o_ref[i] = jnp.maximum(
                y2b_ref[i].astype(f32) * s2 + sh2 + x_ref[i], 0.0)

    xf_spec = pl.BlockSpec((B, C, HW), lambda b: (b, 0, 0))
    ws_spec = pl.BlockSpec((Cout, K9), lambda b: (0, 0))
    mask_spec = pl.BlockSpec((1, HW), lambda b: (0, 0))
    vec_spec = pl.BlockSpec((Cout, 1), lambda b: (0, 0))
    part_spec = pl.BlockSpec((B, Cout, 1), lambda b: (b, 0, 0))
    partfull_spec = pl.BlockSpec((N, Cout, 1), lambda b: (0, 0, 0))
    act_spec = pl.BlockSpec((B, Cout, HW), lambda b: (b, 0, 0))
    part_shape = jax.ShapeDtypeStruct((N, Cout, 1), f32)
    act_shape = jax.ShapeDtypeStruct((N, Cout, HW), bf16)
    parallel = pltpu.CompilerParams(dimension_semantics=("parallel",))

    y1b, ssum1, ssq1 = pl.pallas_call(
        p1_kernel,
        out_shape=(act_shape, part_shape, part_shape),
        grid=(G,),
        in_specs=[xf_spec, ws_spec, mask_spec, mask_spec],
        out_specs=(act_spec, part_spec, part_spec),
        scratch_shapes=[pltpu.VMEM((C, L), bf16), pltpu.VMEM((C, L), bf16),
                        pltpu.VMEM((C, L), bf16), pltpu.VMEM((K9, HW), bf16)],
        compiler_params=parallel,
    )(xf, w1s, mL, mR)

    y2b, ssum2, ssq2 = pl.pallas_call(
        p2_kernel,
        out_shape=(act_shape, part_shape, part_shape),
        grid=(G,),
        in_specs=[act_spec, ws_spec, mask_spec, mask_spec,
                  partfull_spec, partfull_spec, vec_spec, vec_spec],
        out_specs=(act_spec, part_spec, part_spec),
        scratch_shapes=[pltpu.VMEM((Cout, L), bf16),
                        pltpu.VMEM((Cout, L), bf16),
                        pltpu.VMEM((Cout, L), bf16),
                        pltpu.VMEM((K9, HW), bf16)],
        compiler_params=parallel,
    )(y1b, w2s, mL, mR, ssum1, ssq1, g1, b1)

    out = pl.pallas_call(
        p3_kernel,
        out_shape=jax.ShapeDtypeStruct((N, Cout, HW), x.dtype),
        grid=(G,),
        in_specs=[act_spec, xf_spec, partfull_spec, partfull_spec,
                  vec_spec, vec_spec],
        out_specs=pl.BlockSpec((B, Cout, HW), lambda b: (b, 0, 0)),
        compiler_params=parallel,
    )(y2b, xf, ssum2, ssq2, g2, b2)

    return out.reshape(N, Cout, H, W)


def kernel(x, w1, gamma1, beta1, w2, gamma2, beta2):
    return _basic_block(x, w1, gamma1, beta1, w2, gamma2, beta2)


# speedup vs baseline: 1.0627x; 1.0296x over previous
import jax
import jax.numpy as jnp
from jax.experimental import pallas as pl
from jax.experimental.pallas import tpu as pltpu


def _basic_block(x, w1, gamma1, beta1, w2, gamma2, beta2, eps=1e-5):
    N, C, H, W = x.shape
    Cout = w1.shape[0]
    f32 = jnp.float32
    bf16 = jnp.bfloat16
    HW = H * W
    P = W + 1
    L = HW + 2 * P
    K9 = 9 * C
    offs = [kh * W + kw for kh in range(3) for kw in range(3)]
    cnt = float(N * H * W)
    B = 2
    G = N // B

    xf = x.reshape(N, C, HW)
    w1s = jnp.transpose(w1, (0, 2, 3, 1)).reshape(Cout, K9).astype(bf16)
    w2s = jnp.transpose(w2, (0, 2, 3, 1)).reshape(Cout, K9).astype(bf16)
    g1 = gamma1.reshape(Cout, 1).astype(f32)
    b1 = beta1.reshape(Cout, 1).astype(f32)
    g2 = gamma2.reshape(Cout, 1).astype(f32)
    b2 = beta2.reshape(Cout, 1).astype(f32)
    lane = jnp.arange(HW, dtype=jnp.int32) % W
    mL = (lane != W - 1).astype(f32).reshape(1, HW)
    mR = (lane != 0).astype(f32).reshape(1, HW)

    def _tree_sum(v):
        n = v.shape[0]
        while n > 1:
            h = n // 2
            v = v[:h] + v[h:2 * h] if 2 * h == n else \
                jnp.concatenate([v[:h] + v[h:2 * h], v[2 * h:]], axis=0)
            n = v.shape[0]
        return v[0]

    def _fold(ssum_ref, ssq_ref, g_ref, bt_ref):
        mean = _tree_sum(ssum_ref[...]) / cnt
        var = jnp.maximum(_tree_sum(ssq_ref[...]) / cnt - mean * mean, 0.0)
        scale = g_ref[...] / jnp.sqrt(var + eps)
        shift = bt_ref[...] - mean * scale
        return scale, shift

    def fused_kernel(x_ref, w1s_ref, w2s_ref, mL_ref, mR_ref,
                     g1_ref, b1_ref, g2_ref, b2_ref, o_ref,
                     slab_ref, slabL_ref, slabR_ref, xs_ref,
                     y1sc, y2sc, ssum1, ssq1, ssum2, ssq2):
        p = pl.program_id(0)
        b = pl.program_id(1)

        @pl.when(jnp.logical_and(p == 0, b == 0))
        def _():
            for r in (slab_ref, slabL_ref, slabR_ref):
                r[:, :P] = jnp.zeros((C, P), bf16)
                r[:, P + HW:] = jnp.zeros((C, L - P - HW), bf16)

        def _conv(v, w_ref):
            slab_ref[:, P:P + HW] = v.astype(bf16)
            slabL_ref[:, P:P + HW] = (v * mL_ref[...]).astype(bf16)
            slabR_ref[:, P:P + HW] = (v * mR_ref[...]).astype(bf16)
            for k, off in enumerate(offs):
                src = (slabL_ref, slab_ref, slabR_ref)[k % 3]
                xs_ref[k * C:(k + 1) * C, :] = src[:, off:off + HW]
            return jnp.dot(w_ref[...], xs_ref[...],
                           preferred_element_type=f32)

        @pl.when(p == 0)
        def _():
            for i in range(B):
                y1 = _conv(x_ref[i], w1s_ref)
                ssum1[b * B + i] = jnp.sum(y1, axis=1, keepdims=True)
                ssq1[b * B + i] = jnp.sum(y1 * y1, axis=1, keepdims=True)
                y1sc[b * B + i] = y1.astype(bf16)

        @pl.when(p == 1)
        def _():
            s1, sh1 = _fold(ssum1, ssq1, g1_ref, b1_ref)
            for i in range(B):
                a1 = jnp.maximum(y1sc[b * B + i].astype(f32) * s1 + sh1, 0.0)
                y2 = _conv(a1, w2s_ref)
                ssum2[b * B + i] = jnp.sum(y2, axis=1, keepdims=True)
                ssq2[b * B + i] = jnp.sum(y2 * y2, axis=1, keepdims=True)
                y2sc[b * B + i] = y2.astype(bf16)

        @pl.when(p == 2)
        def _():
            s2, sh2 = _fold(ssum2, ssq2, g2_ref, b2_ref)
            for i in range(B):
                o_ref[i] = jnp.maximum(
                    y2sc[b * B + i].astype(f32) * s2 + sh2 + x_ref[i], 0.0)

    xf_spec = pl.BlockSpec(
        (B, C, HW), lambda p, b: (jnp.where(p == 1, G - 1, b), 0, 0))
    ws_spec = pl.BlockSpec((Cout, K9), lambda p, b: (0, 0))
    mask_spec = pl.BlockSpec((1, HW), lambda p, b: (0, 0))
    vec_spec = pl.BlockSpec((Cout, 1), lambda p, b: (0, 0))
    out_spec = pl.BlockSpec(
        (B, Cout, HW), lambda p, b: (jnp.where(p == 2, b, 0), 0, 0))

    out = pl.pallas_call(
        fused_kernel,
        out_shape=jax.ShapeDtypeStruct((N, Cout, HW), x.dtype),
        grid=(3, G),
        in_specs=[xf_spec, ws_spec, ws_spec, mask_spec, mask_spec,
                  vec_spec, vec_spec, vec_spec, vec_spec],
        out_specs=out_spec,
        scratch_shapes=[pltpu.VMEM((C, L), bf16), pltpu.VMEM((C, L), bf16),
                        pltpu.VMEM((C, L), bf16), pltpu.VMEM((K9, HW), bf16),
                        pltpu.VMEM((N, Cout, HW), bf16),
                        pltpu.VMEM((N, Cout, HW), bf16),
                        pltpu.VMEM((N, Cout, 1), f32),
                        pltpu.VMEM((N, Cout, 1), f32),
                        pltpu.VMEM((N, Cout, 1), f32),
                        pltpu.VMEM((N, Cout, 1), f32)],
        compiler_params=pltpu.CompilerParams(
            dimension_semantics=("arbitrary", "arbitrary")),
    )(xf, w1s, w2s, mL, mR, g1, b1, g2, b2)

    return out.reshape(N, Cout, H, W)


def kernel(x, w1, gamma1, beta1, w2, gamma2, beta2):
    return _basic_block(x, w1, gamma1, beta1, w2, gamma2, beta2)


# speedup vs baseline: 1.1114x; 1.0458x over previous
import jax
import jax.numpy as jnp
from jax.experimental import pallas as pl
from jax.experimental.pallas import tpu as pltpu


def _basic_block(x, w1, gamma1, beta1, w2, gamma2, beta2, eps=1e-5):
    N, C, H, W = x.shape
    Cout = w1.shape[0]
    f32 = jnp.float32
    bf16 = jnp.bfloat16
    HW = H * W
    P = W + 1
    L = HW + 2 * P
    K9 = 9 * C
    offs = [kh * W + kw for kh in range(3) for kw in range(3)]
    cnt = float(N * H * W)
    B = 2
    G = N // B

    xf = x.reshape(N, C, HW)
    w1s = jnp.transpose(w1, (0, 2, 3, 1)).reshape(Cout, K9).astype(bf16)
    w2s = jnp.transpose(w2, (0, 2, 3, 1)).reshape(Cout, K9).astype(bf16)
    g1 = gamma1.reshape(Cout, 1).astype(f32)
    b1 = beta1.reshape(Cout, 1).astype(f32)
    g2 = gamma2.reshape(Cout, 1).astype(f32)
    b2 = beta2.reshape(Cout, 1).astype(f32)
    lane = jnp.arange(HW, dtype=jnp.int32) % W
    mL = (lane != W - 1).astype(f32).reshape(1, HW)
    mR = (lane != 0).astype(f32).reshape(1, HW)

    def _tree_sum(v):
        n = v.shape[0]
        while n > 1:
            h = n // 2
            v = v[:h] + v[h:2 * h] if 2 * h == n else \
                jnp.concatenate([v[:h] + v[h:2 * h], v[2 * h:]], axis=0)
            n = v.shape[0]
        return v[0]

    def _fold(ssum_ref, ssq_ref, g_ref, bt_ref):
        mean = _tree_sum(ssum_ref[...]) / cnt
        var = jnp.maximum(_tree_sum(ssq_ref[...]) / cnt - mean * mean, 0.0)
        scale = g_ref[...] / jnp.sqrt(var + eps)
        shift = bt_ref[...] - mean * scale
        return scale, shift

    def fused_kernel(x_ref, w1s_ref, w2s_ref, mL_ref, mR_ref,
                     g1_ref, b1_ref, g2_ref, b2_ref, o_ref,
                     slab_ref, slabL_ref, slabR_ref, xs_ref,
                     y1sc, y2sc, xbsc, ssum1, ssq1, ssum2, ssq2,
                     s1sc, sh1sc, s2sc, sh2sc):
        p = pl.program_id(0)
        b = pl.program_id(1)

        @pl.when(jnp.logical_and(p == 0, b == 0))
        def _():
            for r in (slab_ref, slabL_ref, slabR_ref):
                r[:, :P] = jnp.zeros((C, P), bf16)
                r[:, P + HW:] = jnp.zeros((C, L - P - HW), bf16)

        def _conv(v, w_ref):
            slab_ref[:, P:P + HW] = v.astype(bf16)
            slabL_ref[:, P:P + HW] = (v * mL_ref[...]).astype(bf16)
            slabR_ref[:, P:P + HW] = (v * mR_ref[...]).astype(bf16)
            for k, off in enumerate(offs):
                src = (slabL_ref, slab_ref, slabR_ref)[k % 3]
                xs_ref[k * C:(k + 1) * C, :] = src[:, off:off + HW]
            return jnp.dot(w_ref[...], xs_ref[...],
                           preferred_element_type=f32)

        @pl.when(p == 0)
        def _():
            for i in range(B):
                xbsc[b * B + i] = x_ref[i].astype(bf16)
                y1 = _conv(x_ref[i], w1s_ref)
                ssum1[b * B + i] = jnp.sum(y1, axis=1, keepdims=True)
                ssq1[b * B + i] = jnp.sum(y1 * y1, axis=1, keepdims=True)
                y1sc[b * B + i] = y1.astype(bf16)

        @pl.when(p == 1)
        def _():
            @pl.when(b == 0)
            def _():
                s1, sh1 = _fold(ssum1, ssq1, g1_ref, b1_ref)
                s1sc[...], sh1sc[...] = s1, sh1
            s1, sh1 = s1sc[...], sh1sc[...]
            for i in range(B):
                a1 = jnp.maximum(y1sc[b * B + i].astype(f32) * s1 + sh1, 0.0)
                y2 = _conv(a1, w2s_ref)
                ssum2[b * B + i] = jnp.sum(y2, axis=1, keepdims=True)
                ssq2[b * B + i] = jnp.sum(y2 * y2, axis=1, keepdims=True)
                y2sc[b * B + i] = y2.astype(bf16)

        @pl.when(p == 2)
        def _():
            @pl.when(b == 0)
            def _():
                s2, sh2 = _fold(ssum2, ssq2, g2_ref, b2_ref)
                s2sc[...], sh2sc[...] = s2, sh2
            s2, sh2 = s2sc[...], sh2sc[...]
            for i in range(B):
                o_ref[i] = jnp.maximum(
                    y2sc[b * B + i].astype(f32) * s2 + sh2
                    + xbsc[b * B + i].astype(f32), 0.0)

    xf_spec = pl.BlockSpec(
        (B, C, HW), lambda p, b: (jnp.where(p == 0, b, G - 1), 0, 0))
    ws_spec = pl.BlockSpec((Cout, K9), lambda p, b: (0, 0))
    mask_spec = pl.BlockSpec((1, HW), lambda p, b: (0, 0))
    vec_spec = pl.BlockSpec((Cout, 1), lambda p, b: (0, 0))
    out_spec = pl.BlockSpec(
        (B, Cout, HW), lambda p, b: (jnp.where(p == 2, b, 0), 0, 0))

    out = pl.pallas_call(
        fused_kernel,
        out_shape=jax.ShapeDtypeStruct((N, Cout, HW), x.dtype),
        grid=(3, G),
        in_specs=[xf_spec, ws_spec, ws_spec, mask_spec, mask_spec,
                  vec_spec, vec_spec, vec_spec, vec_spec],
        out_specs=out_spec,
        scratch_shapes=[pltpu.VMEM((C, L), bf16), pltpu.VMEM((C, L), bf16),
                        pltpu.VMEM((C, L), bf16), pltpu.VMEM((K9, HW), bf16),
                        pltpu.VMEM((N, Cout, HW), bf16),
                        pltpu.VMEM((N, Cout, HW), bf16),
                        pltpu.VMEM((N, C, HW), bf16),
                        pltpu.VMEM((N, Cout, 1), f32),
                        pltpu.VMEM((N, Cout, 1), f32),
                        pltpu.VMEM((N, Cout, 1), f32),
                        pltpu.VMEM((N, Cout, 1), f32),
                        pltpu.VMEM((Cout, 1), f32), pltpu.VMEM((Cout, 1), f32),
                        pltpu.VMEM((Cout, 1), f32), pltpu.VMEM((Cout, 1), f32)],
        compiler_params=pltpu.CompilerParams(
            dimension_semantics=("arbitrary", "arbitrary")),
    )(xf, w1s, w2s, mL, mR, g1, b1, g2, b2)

    return out.reshape(N, Cout, H, W)


def kernel(x, w1, gamma1, beta1, w2, gamma2, beta2):
    return _basic_block(x, w1, gamma1, beta1, w2, gamma2, beta2)
